# baseline (speedup 1.0000x reference)
import functools

import jax
import jax.numpy as jnp
from jax.experimental import pallas as pl
from jax.experimental.pallas import tpu as pltpu

_TOPK = 2
_TILE = 256


def _plan_body(rl_ref, d0_ref, d1_ref, w0_ref, w1_ref, te_ref):
    logits = rl_ref[...]
    T, E = logits.shape
    NT = te_ref.shape[0]
    col = jax.lax.broadcasted_iota(jnp.int32, (T, E), 1)
    m1 = jnp.max(logits, axis=1, keepdims=True)
    a1 = jnp.min(jnp.where(logits == m1, col, E), axis=1, keepdims=True)
    masked = jnp.where(col == a1, -jnp.inf, logits)
    m2 = jnp.max(masked, axis=1, keepdims=True)
    a2 = jnp.min(jnp.where(masked == m2, col, E), axis=1, keepdims=True)
    z = jnp.exp(m2 - m1)
    w0_ref[...] = 1.0 / (1.0 + z)
    w1_ref[...] = z / (1.0 + z)

    sel0 = col == a1
    sel1 = col == a2
    M = sel0.astype(jnp.int32) + sel1.astype(jnp.int32)
    x = M
    sh = 1
    while sh < T:
        x = jnp.concatenate(
            [jnp.zeros((sh, E), jnp.int32), x[:-sh, :]], axis=0) + x
        sh *= 2
    excl = x - M
    cnt = x[T - 1:T, :]
    padded = ((cnt + (_TILE - 1)) // _TILE) * _TILE
    r8 = jax.lax.broadcasted_iota(jnp.int32, (E, E), 0)
    c8 = jax.lax.broadcasted_iota(jnp.int32, (E, E), 1)
    U = (r8 < c8).astype(jnp.float32)
    base = jnp.dot(padded.astype(jnp.float32), U,
                   preferred_element_type=jnp.float32).astype(jnp.int32)
    destM = jnp.broadcast_to(base, (T, E)) + excl
    d0_ref[...] = jnp.sum(jnp.where(sel0, destM, 0), axis=1, keepdims=True)
    d1_ref[...] = jnp.sum(jnp.where(sel1, destM, 0), axis=1, keepdims=True)

    jt = jax.lax.broadcasted_iota(jnp.int32, (NT, E), 0)
    endB = jnp.broadcast_to(base + padded, (NT, E))
    s = jnp.sum((jt * _TILE >= endB).astype(jnp.int32), axis=1, keepdims=True)
    te_ref[...] = jnp.minimum(s, E - 1)


def _gemm_body(te_ref, xh_ref, gw_ref, uw_ref, dw_ref, w_ref, yw_ref):
    g = jnp.dot(xh_ref[...], gw_ref[0], preferred_element_type=jnp.float32)
    u = jnp.dot(xh_ref[...], uw_ref[0], preferred_element_type=jnp.float32)
    act = (g * jax.nn.sigmoid(g)) * u
    y = jnp.dot(act.astype(jnp.bfloat16), dw_ref[0],
                preferred_element_type=jnp.float32)
    yw_ref[...] = y * w_ref[...]


@jax.jit
def kernel(hidden_states, router_logits, up_weight, down_weight):
    T, H = hidden_states.shape
    E = up_weight.shape[0]
    I = down_weight.shape[1]
    NT = (T * _TOPK) // _TILE + E
    NP = NT * _TILE

    d0, d1, w0, w1, te = pl.pallas_call(
        _plan_body,
        out_shape=[
            jax.ShapeDtypeStruct((T, 1), jnp.int32),
            jax.ShapeDtypeStruct((T, 1), jnp.int32),
            jax.ShapeDtypeStruct((T, 1), jnp.float32),
            jax.ShapeDtypeStruct((T, 1), jnp.float32),
            jax.ShapeDtypeStruct((NT, 1), jnp.int32),
        ],
    )(router_logits)
    d0 = d0.reshape(T)
    d1 = d1.reshape(T)
    te = te.reshape(NT)

    tok = jnp.arange(T, dtype=jnp.int32)
    src = jnp.zeros((NP,), jnp.int32).at[d0].set(tok).at[d1].set(tok)
    wrow = (jnp.zeros((NP,), jnp.float32)
            .at[d0].set(w0.reshape(T)).at[d1].set(w1.reshape(T)))
    xh = hidden_states.astype(jnp.bfloat16)[src]

    gate_w = up_weight[:, :, :I].astype(jnp.bfloat16)
    up_w = up_weight[:, :, I:].astype(jnp.bfloat16)
    dw = down_weight.astype(jnp.bfloat16)

    yw = pl.pallas_call(
        _gemm_body,
        grid_spec=pltpu.PrefetchScalarGridSpec(
            num_scalar_prefetch=1,
            grid=(NT,),
            in_specs=[
                pl.BlockSpec((_TILE, H), lambda t, te: (t, 0)),
                pl.BlockSpec((1, H, I), lambda t, te: (te[t], 0, 0)),
                pl.BlockSpec((1, H, I), lambda t, te: (te[t], 0, 0)),
                pl.BlockSpec((1, I, H), lambda t, te: (te[t], 0, 0)),
                pl.BlockSpec((_TILE, 1), lambda t, te: (t, 0)),
            ],
            out_specs=pl.BlockSpec((_TILE, H), lambda t, te: (t, 0)),
        ),
        out_shape=jax.ShapeDtypeStruct((NP, H), jnp.float32),
        compiler_params=pltpu.CompilerParams(
            dimension_semantics=("arbitrary",),
        ),
    )(te, xh, gate_w, up_w, dw, wrow.reshape(NP, 1))

    return yw[d0] + yw[d1]
